# baseline (speedup 1.0000x reference)
import jax
import jax.numpy as jnp
from jax import lax
from jax.experimental import pallas as pl
from jax.experimental.pallas import tpu as pltpu
from jax.experimental.pallas import tpu_sc as plsc

B = 16384
F = 26
V = 100000
D = 32

NC = 2
NS = 16
L = 16
NW = NC * NS
RPW = B // NW
RU = 256
CHUNK = 128
NBUF = 3


def _body(tbl_ref, inp_ref, cardb_ref, *refs):
    FS = (len(refs) - 2 - 4 * NBUF) // 2
    outs = refs[:FS]
    idx_v, card_v = refs[FS:FS + 2]
    gidx = refs[FS + 2:FS + 2 + FS]
    rows = refs[FS + 2 + FS:FS + 2 + FS + NBUF]
    tbufs = refs[FS + 2 + FS + NBUF:FS + 2 + FS + 2 * NBUF]
    gsems = refs[FS + 2 + FS + 2 * NBUF:FS + 2 + FS + 3 * NBUF]
    osems = refs[FS + 2 + FS + 3 * NBUF:]
    wid = lax.axis_index("s") * NC + lax.axis_index("c")
    base = wid * RPW

    pltpu.sync_copy(inp_ref.at[:, pl.ds(base, RPW)], idx_v)
    pltpu.sync_copy(cardb_ref, card_v)

    for f in range(FS):
        cvec = card_v[pl.ds(f * L, L)]

        def compute(i, _, cvec=cvec, f=f):
            h = idx_v[f, pl.ds(i * L, L)] + 1
            h = jnp.where(h >= cvec, h - cvec, h)
            gidx[f][pl.ds(i * L, L)] = h
            return 0

        lax.fori_loop(0, RPW // L, compute, 0)

    iota = lax.iota(jnp.int32, L)
    zero_v = jnp.zeros((L,), jnp.int32)
    g_idx = [bl * L + iota for bl in range(CHUNK // L)]

    def transpose_unit(b, c0):
        def tstep(t, _):
            ch = t >> 5
            d = t & (D - 1)
            col = zero_v + d
            a_v = col >> 3
            e_v = col & 7
            c_v = zero_v + ch
            rb = ch * CHUNK
            for bl in range(CHUNK // L):
                vec = plsc.load_gather(rows[b], [rb + g_idx[bl], col])
                plsc.store_scatter(
                    tbufs[b], [a_v, c_v, e_v, g_idx[bl]], vec
                )
            return 0

        lax.fori_loop(0, (RU // CHUNK) * D, tstep, 0)
        return [
            pltpu.async_copy(
                tbufs[b].at[:, pl.ds(ch, 1)],
                outs_slice(c0 + ch),
                osems[b],
            )
            for ch in range(RU // CHUNK)
        ]

    units = [(f, h) for f in range(FS) for h in range(RPW // RU)]
    gcopies = {}
    ocopies = {}
    for u in range(len(units) + 1):
        if u < len(units):
            f, h = units[u]
            b = u % NBUF
            gcopies[u] = [
                pltpu.async_copy(
                    tbl_ref.at[f].at[gidx[f].at[pl.ds(h * RU + j * CHUNK, CHUNK)]],
                    rows[b].at[pl.ds(j * CHUNK, CHUNK)],
                    gsems[b],
                )
                for j in range(RU // CHUNK)
            ]
        if u >= 1:
            g = u - 1
            gf, gh = units[g]
            for c in gcopies.pop(g):
                c.wait()
            if g >= NBUF:
                for c in ocopies.pop(g - NBUF):
                    c.wait()

            def outs_slice(ch_global, gf=gf, gh=gh):
                c = base // CHUNK + gh * (RU // CHUNK) + ch_global
                return outs[gf].at[:, pl.ds(c, 1)]

            ocopies[g] = transpose_unit(g % NBUF, 0)
    for g in sorted(ocopies):
        for c in ocopies.pop(g):
            c.wait()


@jax.jit
def _run(tbl, inp_t, card_b):
    FS = tbl.shape[0]
    mesh = plsc.VectorSubcoreMesh(core_axis_name="c", subcore_axis_name="s")
    fn = pl.kernel(
        _body,
        out_type=tuple(
            jax.ShapeDtypeStruct((D // 8, B // CHUNK, 8, CHUNK), jnp.float32)
            for _ in range(FS)
        ),
        mesh=mesh,
        scratch_types=(
            [
                pltpu.VMEM((FS, RPW), jnp.int32),
                pltpu.VMEM((FS * L,), jnp.int32),
            ]
            + [pltpu.VMEM((RPW,), jnp.int32) for _ in range(FS)]
            + [pltpu.VMEM((RU, D), jnp.float32) for _ in range(NBUF)]
            + [
                pltpu.VMEM((D // 8, RU // CHUNK, 8, CHUNK), jnp.float32)
                for _ in range(NBUF)
            ]
            + [pltpu.SemaphoreType.DMA for _ in range(2 * NBUF)]
        ),
        compiler_params=pltpu.CompilerParams(
            use_tc_tiling_on_sc=False, needs_layout_passes=False
        ),
    )
    return fn(tbl, inp_t, card_b)


NSPLIT = 4


def kernel(inputs, tables, cardinality):
    inp_t = inputs.T
    card_b = jnp.broadcast_to(
        cardinality.astype(jnp.int32)[:, None], (F, L)
    )
    bounds = [(F * s) // NSPLIT for s in range(NSPLIT + 1)]
    outs = []
    for a, b in zip(bounds[:-1], bounds[1:]):
        outs.extend(
            _run(
                tables[a:b],
                inp_t[a:b],
                card_b[a:b].reshape((b - a) * L),
            )
        )
    return tuple(
        x.transpose(1, 3, 0, 2).reshape(B, D) for x in outs
    )


# speedup vs baseline: 1.3054x; 1.3054x over previous
import jax
import jax.numpy as jnp
from jax import lax
from jax.experimental import pallas as pl
from jax.experimental.pallas import tpu as pltpu
from jax.experimental.pallas import tpu_sc as plsc

B = 16384
F = 26
V = 100000
D = 32
DP = 128

NC = 2
NS = 16
L = 16
NW = NC * NS
RPW = B // NW
RU = 256
CHUNK = 128
NBUF = 3


def _body(tbl_ref, inp_ref, cardb_ref, *refs):
    outs = refs[:F]
    idx_v, card_v = refs[F:F + 2]
    gidx = refs[F + 2:F + 2 + F]
    rows = refs[F + 2 + F:F + 2 + F + NBUF]
    gsems = refs[F + 2 + F + NBUF:F + 2 + F + 2 * NBUF]
    osems = refs[F + 2 + F + 2 * NBUF:]
    wid = lax.axis_index("s") * NC + lax.axis_index("c")
    base = wid * RPW

    pltpu.sync_copy(inp_ref.at[:, pl.ds(base, RPW)], idx_v)
    pltpu.sync_copy(cardb_ref, card_v)

    for f in range(F):
        cvec = card_v[pl.ds(f * L, L)]

        def compute(i, _, cvec=cvec, f=f):
            h = idx_v[f, pl.ds(i * L, L)] + 1
            h = jnp.where(h >= cvec, h - cvec, h)
            gidx[f][pl.ds(i * L, L)] = h
            return 0

        lax.fori_loop(0, RPW // L, compute, 0)

    units = [(f, h) for f in range(F) for h in range(RPW // RU)]
    gcopies = {}
    ocopies = {}
    for u in range(len(units) + 1):
        if u < len(units):
            f, h = units[u]
            b = u % NBUF
            if u >= NBUF:
                ocopies.pop(u - NBUF).wait()
            gcopies[u] = [
                pltpu.async_copy(
                    tbl_ref.at[f].at[gidx[f].at[pl.ds(h * RU + j * CHUNK, CHUNK)]],
                    rows[b].at[pl.ds(j * CHUNK, CHUNK)],
                    gsems[b],
                )
                for j in range(RU // CHUNK)
            ]
        if u >= 1:
            g = u - 1
            gf, gh = units[g]
            for c in gcopies.pop(g):
                c.wait()
            ocopies[g] = pltpu.async_copy(
                rows[g % NBUF],
                outs[gf].at[pl.ds(base + gh * RU, RU)],
                osems[g % NBUF],
            )
    for g in sorted(ocopies):
        ocopies.pop(g).wait()


@jax.jit
def _run(tbl, inp_t, card_b):
    mesh = plsc.VectorSubcoreMesh(core_axis_name="c", subcore_axis_name="s")
    fn = pl.kernel(
        _body,
        out_type=tuple(
            jax.ShapeDtypeStruct((B, D), jnp.float32) for _ in range(F)
        ),
        mesh=mesh,
        scratch_types=(
            [
                pltpu.VMEM((F, RPW), jnp.int32),
                pltpu.VMEM((F * L,), jnp.int32),
            ]
            + [pltpu.VMEM((RPW,), jnp.int32) for _ in range(F)]

            + [pltpu.VMEM((RU, D), jnp.float32) for _ in range(NBUF)]
            + [pltpu.SemaphoreType.DMA for _ in range(2 * NBUF)]
        ),
        compiler_params=pltpu.CompilerParams(use_tc_tiling_on_sc=False),
    )
    return fn(tbl, inp_t, card_b)


def kernel(inputs, tables, cardinality):
    inp_t = inputs.T
    card_b = jnp.broadcast_to(
        cardinality.astype(jnp.int32)[:, None], (F, L)
    ).reshape(F * L)
    return tuple(_run(tables, inp_t, card_b))
